# baseline (speedup 1.0000x reference)
import functools

import jax
import jax.numpy as jnp
from jax import lax
from jax.experimental import pallas as pl
from jax.experimental.pallas import tpu as pltpu
from jax.experimental.pallas import tpu_sc as plsc

_LANES = 16


def _gather3_body(b_per_w, nc, tiles_per_row,
                  flat, uid, pid, nid, u_out, p_out, n_out,
                  ids_u, ids_p, ids_n,
                  idx00, idx01, idx02, idx03,
                  idx10, idx11, idx12, idx13,
                  idx20, idx21, idx22, idx23,
                  dat0, dat1, dat2,
                  sem00, sem01, sem02, sem03,
                  sem10, sem11, sem12, sem13,
                  sem20, sem21, sem22, sem23, semw):
    idxs = ((idx00, idx01, idx02, idx03),
            (idx10, idx11, idx12, idx13),
            (idx20, idx21, idx22, idx23))
    sems = ((sem00, sem01, sem02, sem03),
            (sem10, sem11, sem12, sem13),
            (sem20, sem21, sem22, sem23))
    wid = lax.axis_index("s") * nc + lax.axis_index("c")
    base = wid * b_per_w
    gsz = b_per_w * 8

    id_copies = [
        pltpu.async_copy(g_ref.at[pl.ds(base, b_per_w)], ids_v, sem)
        for ids_v, g_ref, sem in ((ids_u, uid, sem00), (ids_p, pid, sem10),
                                  (ids_n, nid, sem20))
    ]

    def make_build(ids_v, idx_w, g):
        goff = g * (tiles_per_row * 1024)

        def build(i, carry):
            r = ids_v[pl.ds(i * _LANES, _LANES)]
            word = ((r >> 7) << 10) + (r & 127) + goff
            dyn = (i >> 3) * 1024 + (i & 7) * _LANES
            for o in range(8):
                idx_w[pl.ds(dyn + o * 128, _LANES)] = word + o * 128
            return carry
        return build

    copies = []
    for k, (ids_v, dat) in enumerate(
            ((ids_u, dat0), (ids_p, dat1), (ids_n, dat2))):
        id_copies[k].wait()
        for g in range(4):
            lax.fori_loop(0, b_per_w // _LANES,
                          make_build(ids_v, idxs[k][g], g), 0, unroll=False)
            copies.append(pltpu.async_copy(
                flat.at[idxs[k][g]], dat.at[pl.ds(g * gsz, gsz)],
                sems[k][g]))

    writes = []
    for g in range(4):
        for k, (dat, o_ref) in enumerate(((dat0, u_out), (dat1, p_out),
                                          (dat2, n_out))):
            copies[k * 4 + g].wait()
            writes.append(pltpu.async_copy(
                dat.at[pl.ds(g * gsz, gsz)],
                o_ref.at[pl.ds(g * (o_ref.shape[0] // 4) + base * 8, gsz)],
                semw))
    for w in writes:
        w.wait()


def _loss_body(u_ref, p_ref, n_ref, out_ref):
    z = jnp.sum(u_ref[...] * (p_ref[...] - n_ref[...]), axis=0,
                keepdims=True)
    zz = -z
    sp = jnp.maximum(zz, 0.0) + jnp.log1p(jnp.exp(-jnp.abs(zz)))
    out_ref[0, 0] = jnp.mean(sp)


def kernel(embedding_table, user_ids, positive_item_ids, negative_item_ids):
    batch = user_ids.shape[0]
    n_rows, dim = embedding_table.shape
    tiles_per_row = n_rows // 128
    info = plsc.get_sparse_core_info()
    nc, ns = info.num_cores, info.num_subcores
    nw = nc * ns
    b_per_w = batch // nw
    mesh = plsc.VectorSubcoreMesh(core_axis_name="c", subcore_axis_name="s")

    table_flat = (embedding_table.T
                  .reshape(dim // 8, 8, tiles_per_row, 128)
                  .transpose(0, 2, 1, 3)
                  .reshape(n_rows * dim))

    out_t = jax.ShapeDtypeStruct((batch * dim,), jnp.float32)
    gather3 = pl.kernel(
        functools.partial(_gather3_body, b_per_w, nc, tiles_per_row),
        out_type=(out_t, out_t, out_t),
        mesh=mesh,
        scratch_types=[
            pltpu.VMEM((b_per_w,), jnp.int32),
            pltpu.VMEM((b_per_w,), jnp.int32),
            pltpu.VMEM((b_per_w,), jnp.int32),
            pltpu.VMEM((8 * b_per_w,), jnp.int32),
            pltpu.VMEM((8 * b_per_w,), jnp.int32),
            pltpu.VMEM((8 * b_per_w,), jnp.int32),
            pltpu.VMEM((8 * b_per_w,), jnp.int32),
            pltpu.VMEM((8 * b_per_w,), jnp.int32),
            pltpu.VMEM((8 * b_per_w,), jnp.int32),
            pltpu.VMEM((8 * b_per_w,), jnp.int32),
            pltpu.VMEM((8 * b_per_w,), jnp.int32),
            pltpu.VMEM((8 * b_per_w,), jnp.int32),
            pltpu.VMEM((8 * b_per_w,), jnp.int32),
            pltpu.VMEM((8 * b_per_w,), jnp.int32),
            pltpu.VMEM((8 * b_per_w,), jnp.int32),
            pltpu.VMEM((dim * b_per_w,), jnp.float32),
            pltpu.VMEM((dim * b_per_w,), jnp.float32),
            pltpu.VMEM((dim * b_per_w,), jnp.float32),
        ] + [pltpu.SemaphoreType.DMA] * 13,
    )
    u_f, p_f, n_f = gather3(
        table_flat,
        user_ids.astype(jnp.int32),
        positive_item_ids.astype(jnp.int32),
        negative_item_ids.astype(jnp.int32),
    )

    def to_t(f):
        return (f.reshape(dim // 8, batch // 128, 8, 128)
                .transpose(0, 2, 1, 3)
                .reshape(dim, batch))

    u_t, p_t, n_t = to_t(u_f), to_t(p_f), to_t(n_f)

    loss = pl.pallas_call(
        _loss_body,
        out_shape=jax.ShapeDtypeStruct((1, 1), jnp.float32),
        out_specs=pl.BlockSpec(memory_space=pltpu.SMEM),
    )(u_t, p_t, n_t)[0, 0]

    return (u_t.T, p_t.T, n_t.T, loss)


# speedup vs baseline: 1.0174x; 1.0174x over previous
import functools

import jax
import jax.numpy as jnp
from jax import lax
from jax.experimental import pallas as pl
from jax.experimental.pallas import tpu as pltpu
from jax.experimental.pallas import tpu_sc as plsc

_LANES = 16


def _gather3_body(b_per_w, nc, tiles_per_row,
                  flat, uid, pid, nid, u_out, p_out, n_out, s_out,
                  ids_u, ids_p, ids_n,
                  idx00, idx01, idx02, idx03,
                  idx10, idx11, idx12, idx13,
                  idx20, idx21, idx22, idx23,
                  dat0, dat1, dat2, sco,
                  sem00, sem01, sem02, sem03,
                  sem10, sem11, sem12, sem13,
                  sem20, sem21, sem22, sem23, semw):
    idxs = ((idx00, idx01, idx02, idx03),
            (idx10, idx11, idx12, idx13),
            (idx20, idx21, idx22, idx23))
    sems = ((sem00, sem01, sem02, sem03),
            (sem10, sem11, sem12, sem13),
            (sem20, sem21, sem22, sem23))
    wid = lax.axis_index("s") * nc + lax.axis_index("c")
    base = wid * b_per_w
    gsz = b_per_w * 8

    id_copies = [
        pltpu.async_copy(g_ref.at[pl.ds(base, b_per_w)], ids_v, sem)
        for ids_v, g_ref, sem in ((ids_u, uid, sem00), (ids_p, pid, sem10),
                                  (ids_n, nid, sem20))
    ]

    def make_build(ids_v, idx_w, g):
        goff = g * (tiles_per_row * 1024)

        def build(i, carry):
            r = ids_v[pl.ds(i * _LANES, _LANES)]
            word = ((r >> 7) << 10) + (r & 127) + goff
            dyn = (i >> 3) * 1024 + (i & 7) * _LANES
            for o in range(8):
                idx_w[pl.ds(dyn + o * 128, _LANES)] = word + o * 128
            return carry
        return build

    copies = []
    for k, (ids_v, dat) in enumerate(
            ((ids_u, dat0), (ids_p, dat1), (ids_n, dat2))):
        id_copies[k].wait()
        for g in range(4):
            lax.fori_loop(0, b_per_w // _LANES,
                          make_build(ids_v, idxs[k][g], g), 0, unroll=False)
            copies.append(pltpu.async_copy(
                flat.at[idxs[k][g]], dat.at[pl.ds(g * gsz, gsz)],
                sems[k][g]))

    writes = []

    def make_score(g):
        def score(i, carry):
            dyn = (i >> 3) * 1024 + (i & 7) * _LANES
            acc = sco[pl.ds(i * _LANES, _LANES)] if g else (
                jnp.zeros((_LANES,), jnp.float32))
            for o in range(8):
                off = pl.ds(dyn + g * gsz + o * 128, _LANES)
                acc += dat0[off] * (dat1[off] - dat2[off])
            sco[pl.ds(i * _LANES, _LANES)] = acc
            return carry
        return score

    for k, (dat, o_ref) in enumerate(((dat0, u_out), (dat1, p_out),
                                      (dat2, n_out))):
        for g in range(4):
            copies[k * 4 + g].wait()
            writes.append(pltpu.async_copy(
                dat.at[pl.ds(g * gsz, gsz)],
                o_ref.at[pl.ds(g * (o_ref.shape[0] // 4) + base * 8, gsz)],
                semw))
            if k == 2:
                lax.fori_loop(0, b_per_w // _LANES, make_score(g), 0,
                              unroll=False)

    writes.append(pltpu.async_copy(sco, s_out.at[pl.ds(base, b_per_w)], semw))
    for w in writes:
        w.wait()


def _loss_body(s_ref, out_ref):
    z = -s_ref[...]
    sp = jnp.maximum(z, 0.0) + jnp.log1p(jnp.exp(-jnp.abs(z)))
    out_ref[0, 0] = jnp.mean(sp)


def kernel(embedding_table, user_ids, positive_item_ids, negative_item_ids):
    batch = user_ids.shape[0]
    n_rows, dim = embedding_table.shape
    tiles_per_row = n_rows // 128
    info = plsc.get_sparse_core_info()
    nc, ns = info.num_cores, info.num_subcores
    nw = nc * ns
    b_per_w = batch // nw
    mesh = plsc.VectorSubcoreMesh(core_axis_name="c", subcore_axis_name="s")

    table_flat = (embedding_table.T
                  .reshape(dim // 8, 8, tiles_per_row, 128)
                  .transpose(0, 2, 1, 3)
                  .reshape(n_rows * dim))

    out_t = jax.ShapeDtypeStruct((batch * dim,), jnp.float32)
    gather3 = pl.kernel(
        functools.partial(_gather3_body, b_per_w, nc, tiles_per_row),
        out_type=(out_t, out_t, out_t,
                  jax.ShapeDtypeStruct((batch,), jnp.float32)),
        mesh=mesh,
        scratch_types=[
            pltpu.VMEM((b_per_w,), jnp.int32),
            pltpu.VMEM((b_per_w,), jnp.int32),
            pltpu.VMEM((b_per_w,), jnp.int32),
            pltpu.VMEM((8 * b_per_w,), jnp.int32),
            pltpu.VMEM((8 * b_per_w,), jnp.int32),
            pltpu.VMEM((8 * b_per_w,), jnp.int32),
            pltpu.VMEM((8 * b_per_w,), jnp.int32),
            pltpu.VMEM((8 * b_per_w,), jnp.int32),
            pltpu.VMEM((8 * b_per_w,), jnp.int32),
            pltpu.VMEM((8 * b_per_w,), jnp.int32),
            pltpu.VMEM((8 * b_per_w,), jnp.int32),
            pltpu.VMEM((8 * b_per_w,), jnp.int32),
            pltpu.VMEM((8 * b_per_w,), jnp.int32),
            pltpu.VMEM((8 * b_per_w,), jnp.int32),
            pltpu.VMEM((8 * b_per_w,), jnp.int32),
            pltpu.VMEM((dim * b_per_w,), jnp.float32),
            pltpu.VMEM((dim * b_per_w,), jnp.float32),
            pltpu.VMEM((dim * b_per_w,), jnp.float32),
            pltpu.VMEM((b_per_w,), jnp.float32),
        ] + [pltpu.SemaphoreType.DMA] * 13,
    )
    u_f, p_f, n_f, scores = gather3(
        table_flat,
        user_ids.astype(jnp.int32),
        positive_item_ids.astype(jnp.int32),
        negative_item_ids.astype(jnp.int32),
    )

    def to_t(f):
        return (f.reshape(dim // 8, batch // 128, 8, 128)
                .transpose(0, 2, 1, 3)
                .reshape(dim, batch))

    u_t, p_t, n_t = to_t(u_f), to_t(p_f), to_t(n_f)

    loss = pl.pallas_call(
        _loss_body,
        out_shape=jax.ShapeDtypeStruct((1, 1), jnp.float32),
        out_specs=pl.BlockSpec(memory_space=pltpu.SMEM),
    )(scores.reshape(batch // 128, 128))[0, 0]

    return (u_t.T, p_t.T, n_t.T, loss)


# speedup vs baseline: 1.0192x; 1.0017x over previous
import functools

import jax
import jax.numpy as jnp
from jax import lax
from jax.experimental import pallas as pl
from jax.experimental.pallas import tpu as pltpu
from jax.experimental.pallas import tpu_sc as plsc

_LANES = 16


def _gather3_body(b_per_w, nc, tiles_per_row,
                  flat, uid, pid, nid, u_out, p_out, n_out, s_out,
                  ids_u, ids_p, ids_n,
                  idx00, idx01, idx02, idx03,
                  idx10, idx11, idx12, idx13,
                  idx20, idx21, idx22, idx23,
                  dat0, dat1, dat2, sco,
                  sem00, sem01, sem02, sem03,
                  sem10, sem11, sem12, sem13,
                  sem20, sem21, sem22, sem23, semw):
    idxs = ((idx00, idx01, idx02, idx03),
            (idx10, idx11, idx12, idx13),
            (idx20, idx21, idx22, idx23))
    sems = ((sem00, sem01, sem02, sem03),
            (sem10, sem11, sem12, sem13),
            (sem20, sem21, sem22, sem23))
    wid = lax.axis_index("s") * nc + lax.axis_index("c")
    base = wid * b_per_w
    gsz = b_per_w * 8

    id_copies = [
        pltpu.async_copy(g_ref.at[pl.ds(base, b_per_w)], ids_v, sem)
        for ids_v, g_ref, sem in ((ids_u, uid, sem00), (ids_p, pid, sem10),
                                  (ids_n, nid, sem20))
    ]

    def make_build(ids_v, idx_w, g):
        goff = g * (tiles_per_row * 1024)

        def build(i, carry):
            r = ids_v[pl.ds(i * _LANES, _LANES)]
            word = ((r >> 7) << 10) + (r & 127) + goff
            dyn = (i >> 3) * 1024 + (i & 7) * _LANES
            for o in range(8):
                idx_w[pl.ds(dyn + o * 128, _LANES)] = word + o * 128
            return carry
        return build

    copies = []
    for k, (ids_v, dat) in enumerate(
            ((ids_u, dat0), (ids_p, dat1), (ids_n, dat2))):
        id_copies[k].wait()
        for g in range(4):
            lax.fori_loop(0, b_per_w // _LANES,
                          make_build(ids_v, idxs[k][g], g), 0, unroll=False)
            copies.append(pltpu.async_copy(
                flat.at[idxs[k][g]], dat.at[pl.ds(g * gsz, gsz)],
                sems[k][g]))

    writes = []

    def make_score(g):
        def score(i, carry):
            dyn = (i >> 3) * 1024 + (i & 7) * _LANES
            acc = sco[pl.ds(i * _LANES, _LANES)] if g else (
                jnp.zeros((_LANES,), jnp.float32))
            for o in range(8):
                off = pl.ds(dyn + g * gsz + o * 128, _LANES)
                acc += dat0[off] * (dat1[off] - dat2[off])
            sco[pl.ds(i * _LANES, _LANES)] = acc
            return carry
        return score

    for g in range(4):
        for k, (dat, o_ref) in enumerate(((dat0, u_out), (dat1, p_out),
                                          (dat2, n_out))):
            copies[k * 4 + g].wait()
            writes.append(pltpu.async_copy(
                dat.at[pl.ds(g * gsz, gsz)],
                o_ref.at[pl.ds(g * (o_ref.shape[0] // 4) + base * 8, gsz)],
                semw))
        lax.fori_loop(0, b_per_w // _LANES, make_score(g), 0, unroll=False)

    writes.append(pltpu.async_copy(sco, s_out.at[pl.ds(base, b_per_w)], semw))
    for w in writes:
        w.wait()


def _loss_body(s_ref, out_ref):
    z = -s_ref[...]
    sp = jnp.maximum(z, 0.0) + jnp.log1p(jnp.exp(-jnp.abs(z)))
    out_ref[0, 0] = jnp.mean(sp)


def kernel(embedding_table, user_ids, positive_item_ids, negative_item_ids):
    batch = user_ids.shape[0]
    n_rows, dim = embedding_table.shape
    tiles_per_row = n_rows // 128
    info = plsc.get_sparse_core_info()
    nc, ns = info.num_cores, info.num_subcores
    nw = nc * ns
    b_per_w = batch // nw
    mesh = plsc.VectorSubcoreMesh(core_axis_name="c", subcore_axis_name="s")

    table_flat = (embedding_table.T
                  .reshape(dim // 8, 8, tiles_per_row, 128)
                  .transpose(0, 2, 1, 3)
                  .reshape(n_rows * dim))

    out_t = jax.ShapeDtypeStruct((batch * dim,), jnp.float32)
    gather3 = pl.kernel(
        functools.partial(_gather3_body, b_per_w, nc, tiles_per_row),
        out_type=(out_t, out_t, out_t,
                  jax.ShapeDtypeStruct((batch,), jnp.float32)),
        mesh=mesh,
        scratch_types=[
            pltpu.VMEM((b_per_w,), jnp.int32),
            pltpu.VMEM((b_per_w,), jnp.int32),
            pltpu.VMEM((b_per_w,), jnp.int32),
            pltpu.VMEM((8 * b_per_w,), jnp.int32),
            pltpu.VMEM((8 * b_per_w,), jnp.int32),
            pltpu.VMEM((8 * b_per_w,), jnp.int32),
            pltpu.VMEM((8 * b_per_w,), jnp.int32),
            pltpu.VMEM((8 * b_per_w,), jnp.int32),
            pltpu.VMEM((8 * b_per_w,), jnp.int32),
            pltpu.VMEM((8 * b_per_w,), jnp.int32),
            pltpu.VMEM((8 * b_per_w,), jnp.int32),
            pltpu.VMEM((8 * b_per_w,), jnp.int32),
            pltpu.VMEM((8 * b_per_w,), jnp.int32),
            pltpu.VMEM((8 * b_per_w,), jnp.int32),
            pltpu.VMEM((8 * b_per_w,), jnp.int32),
            pltpu.VMEM((dim * b_per_w,), jnp.float32),
            pltpu.VMEM((dim * b_per_w,), jnp.float32),
            pltpu.VMEM((dim * b_per_w,), jnp.float32),
            pltpu.VMEM((b_per_w,), jnp.float32),
        ] + [pltpu.SemaphoreType.DMA] * 13,
    )
    u_f, p_f, n_f, scores = gather3(
        table_flat,
        user_ids.astype(jnp.int32),
        positive_item_ids.astype(jnp.int32),
        negative_item_ids.astype(jnp.int32),
    )

    def to_t(f):
        return (f.reshape(dim // 8, batch // 128, 8, 128)
                .transpose(0, 2, 1, 3)
                .reshape(dim, batch))

    u_t, p_t, n_t = to_t(u_f), to_t(p_f), to_t(n_f)

    loss = pl.pallas_call(
        _loss_body,
        out_shape=jax.ShapeDtypeStruct((1, 1), jnp.float32),
        out_specs=pl.BlockSpec(memory_space=pltpu.SMEM),
    )(scores.reshape(batch // 128, 128))[0, 0]

    return (u_t.T, p_t.T, n_t.T, loss)
